# Initial kernel scaffold; baseline (speedup 1.0000x reference)
import jax
import jax.numpy as jnp
from jax.experimental import pallas as pl


def kernel(x, edge_index, batch, W1, theta1, b1, W2, theta2, b2, lin1_W, lin1_b, lin2_W, lin2_b):
    raise NotImplementedError("write your pallas kernel here")


# speedup vs baseline: 24.9441x; 24.9441x over previous
import functools
import jax
import jax.numpy as jnp
from jax.experimental import pallas as pl
from jax.experimental.pallas import tpu as pltpu

_N = 10000
_E = 320000
_K = 14
_H = 8
_HID = 64
_ROWS = 1000


def _matmul_theta_body(x_ref, w_ref, th_ref, o_ref, *, F):
    x = x_ref[...]
    for k in range(_K):
        wk = th_ref[k, 0] * w_ref[:, 0:_HID]
        for h in range(1, _H):
            wk = wk + th_ref[k, h] * w_ref[:, h * _HID:(h + 1) * _HID]
        o_ref[:, k * _HID:(k + 1) * _HID] = jnp.dot(
            x, wk * (1.0 / _H), preferred_element_type=jnp.float32)


def _matmul_theta(h_in, W, theta):
    F = h_in.shape[1]
    grid = _N // _ROWS
    return pl.pallas_call(
        functools.partial(_matmul_theta_body, F=F),
        grid=(grid,),
        in_specs=[
            pl.BlockSpec((_ROWS, F), lambda i: (i, 0)),
            pl.BlockSpec((F, _H * _HID), lambda i: (0, 0)),
            pl.BlockSpec((_K, _H), lambda i: (0, 0)),
        ],
        out_specs=pl.BlockSpec((_ROWS, _K * _HID), lambda i: (i, 0)),
        out_shape=jax.ShapeDtypeStruct((_N, _K * _HID), jnp.float32),
    )(h_in, W, theta)


def _head_body(g_ref, w1_ref, b1_ref, w2_ref, b2_ref, o_ref):
    g = g_ref[...]
    g = jnp.maximum(jnp.dot(g, w1_ref[...], preferred_element_type=jnp.float32)
                    + b1_ref[...], 0.0)
    logits = jnp.dot(g, w2_ref[...], preferred_element_type=jnp.float32) + b2_ref[...]
    m = jnp.max(logits, axis=1, keepdims=True)
    s = logits - m
    lse = jnp.log(jnp.sum(jnp.exp(s), axis=1, keepdims=True))
    o_ref[...] = s - lse


def _head(g, lin1_W, lin1_b, lin2_W, lin2_b):
    C = lin2_W.shape[1]
    return pl.pallas_call(
        _head_body,
        out_shape=jax.ShapeDtypeStruct((1, C), jnp.float32),
    )(g, lin1_W, lin1_b.reshape(1, -1), lin2_W, lin2_b.reshape(1, -1))


def kernel(x, edge_index, batch, W1, theta1, b1, W2, theta2, b2,
           lin1_W, lin1_b, lin2_W, lin2_b):
    row = edge_index[0]
    col = edge_index[1]
    deg = jax.ops.segment_sum(jnp.ones((_E,), jnp.float32), row, num_segments=_N)
    dinv = jnp.where(deg > 0, deg ** -0.5, 0.0)

    def S(c):
        return jax.ops.segment_sum(jnp.take(c, col, axis=0), row, num_segments=_N)

    def layer(h_in, W, theta, b):
        Y = _matmul_theta(h_in, W, theta)
        bk1 = jnp.zeros((_N, _HID), jnp.float32)
        bk2 = jnp.zeros((_N, _HID), jnp.float32)
        for k in range(_K - 1, 0, -1):
            Lb = -dinv[:, None] * S(dinv[:, None] * bk1)
            bk = Y[:, k * _HID:(k + 1) * _HID] + 2.0 * Lb - bk2
            bk1, bk2 = bk, bk1
        Lb = -dinv[:, None] * S(dinv[:, None] * bk1)
        out = Y[:, 0:_HID] + Lb - bk2
        return out + b

    h = jax.nn.relu(layer(x, W1, theta1, b1))
    h = jax.nn.relu(layer(h, W2, theta2, b2))
    g = h.mean(axis=0, keepdims=True)
    return _head(g, lin1_W, lin1_b, lin2_W, lin2_b)


# speedup vs baseline: 245.8384x; 9.8556x over previous
import functools
import jax
import jax.numpy as jnp
from jax import lax
from jax.experimental import pallas as pl
from jax.experimental.pallas import tpu as pltpu
from jax.experimental.pallas import tpu_sc as plsc

_N = 10000
_E = 320000
_K = 14
_H = 8
_HID = 64
_HALF = 32
_NT = 16
_EPT = _E // _NT
_CHW = 128
_SCH = 8
_NCH = 160
_NSCH = _NCH // _SCH
_EPAD = _NCH * _CHW - _EPT
_NPT = 10240
_RPT = _NPT // _NT
_SUBR = 128
_NSUB = _RPT // _SUBR
_ROWS = 1000


def _mm_body(x_ref, w_ref, th_ref, o_ref, *, F, two_d_in):
    if two_d_in:
        x = jnp.concatenate([x_ref[0], x_ref[1]], axis=1)
    else:
        x = x_ref[...]
    wk = th_ref[0, 0, 0] * w_ref[:, 0:_HID]
    for h in range(1, _H):
        wk = wk + th_ref[0, 0, h] * w_ref[:, h * _HID:(h + 1) * _HID]
    y = jnp.dot(x, wk * (1.0 / _H), preferred_element_type=jnp.float32)
    o_ref[0, 0] = y[:, :_HALF]
    o_ref[0, 1] = y[:, _HALF:]


def _matmul_theta(h_in, W, theta):
    two_d = h_in.ndim == 3
    F = W.shape[0]
    if two_d:
        x_spec = pl.BlockSpec((2, _ROWS, _HALF), lambda k, i: (0, i, 0))
    else:
        x_spec = pl.BlockSpec((_ROWS, F), lambda k, i: (i, 0))
    return pl.pallas_call(
        functools.partial(_mm_body, F=F, two_d_in=two_d),
        grid=(_K, _N // _ROWS),
        in_specs=[
            x_spec,
            pl.BlockSpec((F, _H * _HID), lambda k, i: (0, 0)),
            pl.BlockSpec((1, 1, _H), lambda k, i: (k, 0, 0),
                         memory_space=pltpu.SMEM),
        ],
        out_specs=pl.BlockSpec((1, 2, _ROWS, _HALF), lambda k, i: (k, 0, i, 0)),
        out_shape=jax.ShapeDtypeStruct((_K, 2, _NPT, _HALF), jnp.float32),
    )(h_in, W, theta.reshape(_K, 1, _H))


def _head_body(h_ref, w1_ref, b1_ref, w2_ref, b2_ref, o_ref):
    g0 = jnp.sum(h_ref[0], axis=0, keepdims=True)
    g1 = jnp.sum(h_ref[1], axis=0, keepdims=True)
    g = jnp.concatenate([g0, g1], axis=1) * (1.0 / _N)
    g = jnp.maximum(jnp.dot(g, w1_ref[...], preferred_element_type=jnp.float32)
                    + b1_ref[...], 0.0)
    logits = jnp.dot(g, w2_ref[...], preferred_element_type=jnp.float32) + b2_ref[...]
    m = jnp.max(logits, axis=1, keepdims=True)
    s = logits - m
    o_ref[...] = s - jnp.log(jnp.sum(jnp.exp(s), axis=1, keepdims=True))


def _head(h2, lin1_W, lin1_b, lin2_W, lin2_b):
    C = lin2_W.shape[1]
    return pl.pallas_call(
        _head_body,
        out_shape=jax.ShapeDtypeStruct((1, C), jnp.float32),
    )(h2, lin1_W, lin1_b.reshape(1, -1), lin2_W, lin2_b.reshape(1, -1))


def _rsqrt_pos(d):
    i = lax.bitcast_convert_type(d, jnp.int32)
    i = jnp.int32(0x5F3759DF) - lax.shift_right_arithmetic(i, 1)
    y = lax.bitcast_convert_type(i, jnp.float32)
    for _ in range(4):
        y = y * (1.5 - 0.5 * d * y * y)
    return jnp.where(d > 0.5, y, 0.0)


def _sc_layer_body(first_layer, *refs):
    if first_layer:
        (y_hbm, ep_hbm, bias_hbm,
         h_out, dinv_out,
         CT, ACC,
         sbuf, gbuf, accb, ctb, ysub, dinvt, b0t, b1t, bb,
         ysem) = refs
    else:
        (y_hbm, ep_hbm, bias_hbm, dinv_hbm,
         h_out,
         CT, ACC,
         sbuf, gbuf, accb, ctb, ysub, dinvt, b0t, b1t, bb,
         ysem) = refs

    c = lax.axis_index("c")
    wid = lax.axis_index("s")
    row0 = wid * _RPT

    def vloop(n, body):
        lax.fori_loop(0, n, lambda i, _: (body(i), 0)[1], 0)

    def rh(i):
        return lax.div(i, jnp.int32(2)), pl.ds(16 * lax.rem(i, jnp.int32(2)), 16)

    def zero_accb(i):
        r, h16 = rh(i)
        accb[r, h16] = jnp.zeros((16,), jnp.float32)

    pltpu.sync_copy(bias_hbm.at[c], bb)
    vloop(2 * _SUBR, zero_accb)
    for s in range(_NSUB):
        pltpu.sync_copy(accb, ACC.at[pl.ds(row0 + s * _SUBR, _SUBR), :])
    if not first_layer:
        pltpu.sync_copy(dinv_hbm.at[pl.ds(row0, _RPT), :], dinvt)
    if first_layer:
        def fill_ones(i):
            gbuf[lax.div(i, jnp.int32(2)), pl.ds(16 * lax.rem(i, jnp.int32(2)), 16)] = (
                jnp.ones((16,), jnp.float32))
        vloop(2 * _CHW, fill_ones)
    plsc.subcore_barrier()

    if first_layer:
        def deg_super(g, _):
            pltpu.sync_copy(ep_hbm.at[wid, pl.ds(g * _SCH, _SCH)], sbuf)
            for jj in range(_SCH):
                pltpu.sync_copy(gbuf, ACC.at[sbuf.at[jj, 0]], add=True)
            return 0
        lax.fori_loop(0, _NSCH, deg_super, 0)
        plsc.subcore_barrier()

        def dinv_sub(s, _):
            r0 = row0 + s * _SUBR
            pltpu.sync_copy(ACC.at[pl.ds(r0, _SUBR), :], accb)

            def dinv_row(i):
                r, h16 = rh(i)
                dinvt[s * _SUBR + r, h16] = _rsqrt_pos(accb[r, h16])
                accb[r, h16] = jnp.zeros((16,), jnp.float32)
            vloop(2 * _SUBR, dinv_row)
            pltpu.sync_copy(accb, ACC.at[pl.ds(r0, _SUBR), :])
            return 0
        lax.fori_loop(0, _NSUB, dinv_sub, 0)

        @pl.when(c == 0)
        def _():
            pltpu.sync_copy(dinvt, dinv_out.at[pl.ds(row0, _RPT), :])

    def init_sub(s, _):
        r0 = row0 + s * _SUBR
        pltpu.sync_copy(y_hbm.at[_K - 1, c, pl.ds(r0, _SUBR), :], ysub)

        def init_row(i):
            r, h16 = rh(i)
            yv = ysub[r, h16]
            fl = pl.ds((s * _SUBR + r) * _HALF + 16 * lax.rem(i, jnp.int32(2)), 16)
            b1t[fl] = yv
            b0t[fl] = jnp.zeros((16,), jnp.float32)
            ctb[r, h16] = dinvt[s * _SUBR + r, h16] * yv
        vloop(2 * _SUBR, init_row)
        pltpu.sync_copy(ctb, CT.at[pl.ds(r0, _SUBR), :])
        return 0
    lax.fori_loop(0, _NSUB, init_sub, 0)
    plsc.subcore_barrier()

    def step(k, bprev, final):
        def super_chunk(g, _):
            pltpu.sync_copy(ep_hbm.at[wid, pl.ds(g * _SCH, _SCH)], sbuf)
            for jj in range(_SCH):
                pltpu.sync_copy(CT.at[sbuf.at[jj, 1]], gbuf)
                pltpu.sync_copy(gbuf, ACC.at[sbuf.at[jj, 0]], add=True)
            return 0
        lax.fori_loop(0, _NSCH, super_chunk, 0)
        plsc.subcore_barrier()

        def ew_sub(s, _):
            r0 = row0 + s * _SUBR
            pltpu.sync_copy(y_hbm.at[k, c, pl.ds(r0, _SUBR), :], ysub)
            pltpu.sync_copy(ACC.at[pl.ds(r0, _SUBR), :], accb)

            def ew_row(i):
                r, h16 = rh(i)
                fl = pl.ds((s * _SUBR + r) * _HALF + 16 * lax.rem(i, jnp.int32(2)), 16)
                a = accb[r, h16]
                d = dinvt[s * _SUBR + r, h16]
                yv = ysub[r, h16]
                if final:
                    out = yv - d * a - bprev[fl] + bb[pl.ds(16 * lax.rem(i, jnp.int32(2)), 16)]
                    ctb[r, h16] = jnp.maximum(out, 0.0)
                else:
                    bnew = yv - 2.0 * (d * a) - bprev[fl]
                    bprev[fl] = bnew
                    ctb[r, h16] = d * bnew
                    accb[r, h16] = jnp.zeros((16,), jnp.float32)
            vloop(2 * _SUBR, ew_row)

            if final:
                pltpu.sync_copy(ctb, h_out.at[c, pl.ds(r0, _SUBR), :])
            else:
                pltpu.sync_copy(ctb, CT.at[pl.ds(r0, _SUBR), :])
                pltpu.sync_copy(accb, ACC.at[pl.ds(r0, _SUBR), :])
            return 0
        lax.fori_loop(0, _NSUB, ew_sub, 0)
        if not final:
            plsc.subcore_barrier()

    def double_step(t, _):
        step(jnp.int32(12) - 2 * t, b0t, False)
        step(jnp.int32(11) - 2 * t, b1t, False)
        return 0
    lax.fori_loop(0, 6, double_step, 0)

    step(jnp.int32(0), b0t, True)

    @pl.when(wid == _NT - 1)
    def _():
        vloop(2 * _SUBR, zero_accb)
        pltpu.sync_copy(accb, h_out.at[c, pl.ds(_N, _SUBR), :])
        pltpu.sync_copy(accb.at[pl.ds(0, _NPT - _N - _SUBR), :],
                        h_out.at[c, pl.ds(_N + _SUBR, _NPT - _N - _SUBR), :])


def _sc_layer(first_layer):
    out_type = [jax.ShapeDtypeStruct((2, _NPT, _HALF), jnp.float32)]
    if first_layer:
        out_type.append(jax.ShapeDtypeStruct((_NPT, _HALF), jnp.float32))
    return functools.partial(
        pl.kernel,
        functools.partial(_sc_layer_body, first_layer),
        out_type=tuple(out_type),
        mesh=plsc.VectorSubcoreMesh(core_axis_name="c", subcore_axis_name="s"),
        compiler_params=pltpu.CompilerParams(use_tc_tiling_on_sc=False),
        scratch_types=(
            pltpu.VMEM_SHARED((_NPT, _HALF), jnp.float32),
            pltpu.VMEM_SHARED((_NPT, _HALF), jnp.float32),
            pltpu.VMEM((_SCH, 2, _CHW), jnp.int32),
            pltpu.VMEM((_CHW, _HALF), jnp.float32),
            pltpu.VMEM((_SUBR, _HALF), jnp.float32),
            pltpu.VMEM((_SUBR, _HALF), jnp.float32),
            pltpu.VMEM((_SUBR, _HALF), jnp.float32),
            pltpu.VMEM((_RPT, _HALF), jnp.float32),
            pltpu.VMEM((_RPT * _HALF,), jnp.float32),
            pltpu.VMEM((_RPT * _HALF,), jnp.float32),
            pltpu.VMEM((_HALF,), jnp.float32),
            pltpu.SemaphoreType.DMA,
        ),
    )()


def kernel(x, edge_index, batch, W1, theta1, b1, W2, theta2, b2,
           lin1_W, lin1_b, lin2_W, lin2_b):
    ei = edge_index.reshape(2, _NT, _EPT)
    padv = jnp.full((2, _NT, _EPAD), _N, dtype=jnp.int32)
    ep = jnp.concatenate([ei, padv], axis=2).reshape(2, _NT, _NCH, _CHW)
    ep = jnp.transpose(ep, (1, 2, 0, 3))

    y1 = _matmul_theta(x, W1, theta1)
    h1, dinv = _sc_layer(True)(y1, ep, b1.reshape(2, _HALF))
    y2 = _matmul_theta(h1, W2, theta2)
    (h2,) = _sc_layer(False)(y2, ep, b2.reshape(2, _HALF), dinv)
    return _head(h2, lin1_W, lin1_b, lin2_W, lin2_b)


# speedup vs baseline: 299.3349x; 1.2176x over previous
import functools
import jax
import jax.numpy as jnp
from jax import lax
from jax.experimental import pallas as pl
from jax.experimental.pallas import tpu as pltpu
from jax.experimental.pallas import tpu_sc as plsc

_N = 10000
_E = 320000
_K = 14
_H = 8
_HID = 64
_HALF = 32
_NT = 16
_EPT = _E // _NT
_CHW = 128
_SCH = 8
_NCH = 160
_NSCH = _NCH // _SCH
_EPAD = _NCH * _CHW - _EPT
_NPT = 10240
_RPT = _NPT // _NT
_SUBR = 128
_NSUB = _RPT // _SUBR
_ROWS = 1000


def _mm_body(x_ref, w_ref, th_ref, o_ref, *, F, two_d_in):
    if two_d_in:
        x = jnp.concatenate([x_ref[0], x_ref[1]], axis=1)
    else:
        x = x_ref[...]
    wk = th_ref[0, 0, 0] * w_ref[:, 0:_HID]
    for h in range(1, _H):
        wk = wk + th_ref[0, 0, h] * w_ref[:, h * _HID:(h + 1) * _HID]
    y = jnp.dot(x, wk * (1.0 / _H), preferred_element_type=jnp.float32)
    o_ref[0, 0] = y[:, :_HALF]
    o_ref[0, 1] = y[:, _HALF:]


def _matmul_theta(h_in, W, theta):
    two_d = h_in.ndim == 3
    F = W.shape[0]
    if two_d:
        x_spec = pl.BlockSpec((2, _ROWS, _HALF), lambda k, i: (0, i, 0))
    else:
        x_spec = pl.BlockSpec((_ROWS, F), lambda k, i: (i, 0))
    return pl.pallas_call(
        functools.partial(_mm_body, F=F, two_d_in=two_d),
        grid=(_K, _N // _ROWS),
        in_specs=[
            x_spec,
            pl.BlockSpec((F, _H * _HID), lambda k, i: (0, 0)),
            pl.BlockSpec((1, 1, _H), lambda k, i: (k, 0, 0),
                         memory_space=pltpu.SMEM),
        ],
        out_specs=pl.BlockSpec((1, 2, _ROWS, _HALF), lambda k, i: (k, 0, i, 0)),
        out_shape=jax.ShapeDtypeStruct((_K, 2, _NPT, _HALF), jnp.float32),
    )(h_in, W, theta.reshape(_K, 1, _H))


def _head_body(h_ref, w1_ref, b1_ref, w2_ref, b2_ref, o_ref):
    g0 = jnp.sum(h_ref[0], axis=0, keepdims=True)
    g1 = jnp.sum(h_ref[1], axis=0, keepdims=True)
    g = jnp.concatenate([g0, g1], axis=1) * (1.0 / _N)
    g = jnp.maximum(jnp.dot(g, w1_ref[...], preferred_element_type=jnp.float32)
                    + b1_ref[...], 0.0)
    logits = jnp.dot(g, w2_ref[...], preferred_element_type=jnp.float32) + b2_ref[...]
    m = jnp.max(logits, axis=1, keepdims=True)
    s = logits - m
    o_ref[...] = s - jnp.log(jnp.sum(jnp.exp(s), axis=1, keepdims=True))


def _head(h2, lin1_W, lin1_b, lin2_W, lin2_b):
    C = lin2_W.shape[1]
    return pl.pallas_call(
        _head_body,
        out_shape=jax.ShapeDtypeStruct((1, C), jnp.float32),
    )(h2, lin1_W, lin1_b.reshape(1, -1), lin2_W, lin2_b.reshape(1, -1))


def _rsqrt_pos(d):
    i = lax.bitcast_convert_type(d, jnp.int32)
    i = jnp.int32(0x5F3759DF) - lax.shift_right_arithmetic(i, 1)
    y = lax.bitcast_convert_type(i, jnp.float32)
    for _ in range(4):
        y = y * (1.5 - 0.5 * d * y * y)
    return jnp.where(d > 0.5, y, 0.0)


def _sc_layer_body(first_layer, *refs):
    if first_layer:
        (y_hbm, ep_hbm, bias_hbm,
         h_out, dinv_out,
         CT, ACC,
         sbuf, gbuf, gbuf2, accb, ctb, ysub, dinvt, b0t, b1t, bb,
         ysem, ssem) = refs
    else:
        (y_hbm, ep_hbm, bias_hbm, dinv_hbm,
         h_out,
         CT, ACC,
         sbuf, gbuf, gbuf2, accb, ctb, ysub, dinvt, b0t, b1t, bb,
         ysem, ssem) = refs

    c = lax.axis_index("c")
    wid = lax.axis_index("s")
    row0 = wid * _RPT

    def vloop(n, body):
        lax.fori_loop(0, n, lambda i, _: (body(i), 0)[1], 0)

    def rh(i):
        return lax.div(i, jnp.int32(2)), pl.ds(16 * lax.rem(i, jnp.int32(2)), 16)

    def zero_accb(i):
        r, h16 = rh(i)
        accb[r, h16] = jnp.zeros((16,), jnp.float32)

    pltpu.sync_copy(bias_hbm.at[c], bb)
    vloop(2 * _SUBR, zero_accb)
    for s in range(_NSUB):
        pltpu.sync_copy(accb, ACC.at[pl.ds(row0 + s * _SUBR, _SUBR), :])
    if not first_layer:
        pltpu.sync_copy(dinv_hbm.at[pl.ds(row0, _RPT), :], dinvt)
    if first_layer:
        def fill_ones(i):
            gbuf[lax.div(i, jnp.int32(2)), pl.ds(16 * lax.rem(i, jnp.int32(2)), 16)] = (
                jnp.ones((16,), jnp.float32))
        vloop(2 * _CHW, fill_ones)
    plsc.subcore_barrier()

    if first_layer:
        def deg_super(g, _):
            pltpu.sync_copy(ep_hbm.at[wid, pl.ds(g * _SCH, _SCH)], sbuf)
            for jj in range(_SCH):
                pltpu.sync_copy(gbuf, ACC.at[sbuf.at[jj, 0]], add=True)
            return 0
        lax.fori_loop(0, _NSCH, deg_super, 0)
        plsc.subcore_barrier()

        def dinv_sub(s, _):
            r0 = row0 + s * _SUBR
            pltpu.sync_copy(ACC.at[pl.ds(r0, _SUBR), :], accb)

            def dinv_row(i):
                r, h16 = rh(i)
                dinvt[s * _SUBR + r, h16] = _rsqrt_pos(accb[r, h16])
                accb[r, h16] = jnp.zeros((16,), jnp.float32)
            vloop(2 * _SUBR, dinv_row)
            pltpu.sync_copy(accb, ACC.at[pl.ds(r0, _SUBR), :])
            return 0
        lax.fori_loop(0, _NSUB, dinv_sub, 0)

        @pl.when(c == 0)
        def _():
            pltpu.sync_copy(dinvt, dinv_out.at[pl.ds(row0, _RPT), :])

    def init_sub(s, _):
        r0 = row0 + s * _SUBR
        pltpu.sync_copy(y_hbm.at[_K - 1, c, pl.ds(r0, _SUBR), :], ysub)

        def init_row(i):
            r, h16 = rh(i)
            yv = ysub[r, h16]
            fl = pl.ds((s * _SUBR + r) * _HALF + 16 * lax.rem(i, jnp.int32(2)), 16)
            b1t[fl] = yv
            b0t[fl] = jnp.zeros((16,), jnp.float32)
            ctb[r, h16] = dinvt[s * _SUBR + r, h16] * yv
        vloop(2 * _SUBR, init_row)
        pltpu.sync_copy(ctb, CT.at[pl.ds(r0, _SUBR), :])
        return 0
    lax.fori_loop(0, _NSUB, init_sub, 0)
    plsc.subcore_barrier()

    def wait_scatter():
        pltpu.make_async_copy(gbuf, ACC.at[sbuf.at[0, 0]], ssem).wait()

    def step(k, bprev, final):
        def super_chunk(g, _):
            pltpu.sync_copy(ep_hbm.at[wid, pl.ds(g * _SCH, _SCH)], sbuf)
            for jj in range(_SCH):
                b = gbuf if jj % 2 == 0 else gbuf2
                pltpu.sync_copy(CT.at[sbuf.at[jj, 1]], b)
                if jj >= 1:
                    wait_scatter()
                pltpu.async_copy(b, ACC.at[sbuf.at[jj, 0]], ssem, add=True)
            wait_scatter()
            return 0
        lax.fori_loop(0, _NSCH, super_chunk, 0)
        plsc.subcore_barrier()

        def ew_sub(s, _):
            r0 = row0 + s * _SUBR
            pltpu.sync_copy(y_hbm.at[k, c, pl.ds(r0, _SUBR), :], ysub)
            pltpu.sync_copy(ACC.at[pl.ds(r0, _SUBR), :], accb)

            def ew_row(i):
                r, h16 = rh(i)
                fl = pl.ds((s * _SUBR + r) * _HALF + 16 * lax.rem(i, jnp.int32(2)), 16)
                a = accb[r, h16]
                d = dinvt[s * _SUBR + r, h16]
                yv = ysub[r, h16]
                if final:
                    out = yv - d * a - bprev[fl] + bb[pl.ds(16 * lax.rem(i, jnp.int32(2)), 16)]
                    ctb[r, h16] = jnp.maximum(out, 0.0)
                else:
                    bnew = yv - 2.0 * (d * a) - bprev[fl]
                    bprev[fl] = bnew
                    ctb[r, h16] = d * bnew
                    accb[r, h16] = jnp.zeros((16,), jnp.float32)
            vloop(2 * _SUBR, ew_row)

            if final:
                pltpu.sync_copy(ctb, h_out.at[c, pl.ds(r0, _SUBR), :])
            else:
                pltpu.sync_copy(ctb, CT.at[pl.ds(r0, _SUBR), :])
                pltpu.sync_copy(accb, ACC.at[pl.ds(r0, _SUBR), :])
            return 0
        lax.fori_loop(0, _NSUB, ew_sub, 0)
        if not final:
            plsc.subcore_barrier()

    def double_step(t, _):
        step(jnp.int32(12) - 2 * t, b0t, False)
        step(jnp.int32(11) - 2 * t, b1t, False)
        return 0
    lax.fori_loop(0, 6, double_step, 0)

    step(jnp.int32(0), b0t, True)

    @pl.when(wid == _NT - 1)
    def _():
        vloop(2 * _SUBR, zero_accb)
        pltpu.sync_copy(accb, h_out.at[c, pl.ds(_N, _SUBR), :])
        pltpu.sync_copy(accb.at[pl.ds(0, _NPT - _N - _SUBR), :],
                        h_out.at[c, pl.ds(_N + _SUBR, _NPT - _N - _SUBR), :])


def _sc_layer(first_layer):
    out_type = [jax.ShapeDtypeStruct((2, _NPT, _HALF), jnp.float32)]
    if first_layer:
        out_type.append(jax.ShapeDtypeStruct((_NPT, _HALF), jnp.float32))
    return functools.partial(
        pl.kernel,
        functools.partial(_sc_layer_body, first_layer),
        out_type=tuple(out_type),
        mesh=plsc.VectorSubcoreMesh(core_axis_name="c", subcore_axis_name="s"),
        compiler_params=pltpu.CompilerParams(use_tc_tiling_on_sc=False),
        scratch_types=(
            pltpu.VMEM_SHARED((_NPT, _HALF), jnp.float32),
            pltpu.VMEM_SHARED((_NPT, _HALF), jnp.float32),
            pltpu.VMEM((_SCH, 2, _CHW), jnp.int32),
            pltpu.VMEM((_CHW, _HALF), jnp.float32),
            pltpu.VMEM((_CHW, _HALF), jnp.float32),
            pltpu.VMEM((_SUBR, _HALF), jnp.float32),
            pltpu.VMEM((_SUBR, _HALF), jnp.float32),
            pltpu.VMEM((_SUBR, _HALF), jnp.float32),
            pltpu.VMEM((_RPT, _HALF), jnp.float32),
            pltpu.VMEM((_RPT * _HALF,), jnp.float32),
            pltpu.VMEM((_RPT * _HALF,), jnp.float32),
            pltpu.VMEM((_HALF,), jnp.float32),
            pltpu.SemaphoreType.DMA,
            pltpu.SemaphoreType.DMA,
        ),
    )()


def kernel(x, edge_index, batch, W1, theta1, b1, W2, theta2, b2,
           lin1_W, lin1_b, lin2_W, lin2_b):
    ei = edge_index.reshape(2, _NT, _EPT)
    padv = jnp.full((2, _NT, _EPAD), _N, dtype=jnp.int32)
    ep = jnp.concatenate([ei, padv], axis=2).reshape(2, _NT, _NCH, _CHW)
    ep = jnp.transpose(ep, (1, 2, 0, 3))

    y1 = _matmul_theta(x, W1, theta1)
    h1, dinv = _sc_layer(True)(y1, ep, b1.reshape(2, _HALF))
    y2 = _matmul_theta(h1, W2, theta2)
    (h2,) = _sc_layer(False)(y2, ep, b2.reshape(2, _HALF), dinv)
    return _head(h2, lin1_W, lin1_b, lin2_W, lin2_b)


# speedup vs baseline: 307.1848x; 1.0262x over previous
import functools
import jax
import jax.numpy as jnp
from jax import lax
from jax.experimental import pallas as pl
from jax.experimental.pallas import tpu as pltpu
from jax.experimental.pallas import tpu_sc as plsc

_N = 10000
_E = 320000
_K = 14
_H = 8
_HID = 64
_HALF = 32
_NT = 16
_EPT = _E // _NT
_CHW = 128
_SCH = 8
_NCH = 160
_NSCH = _NCH // _SCH
_EPAD = _NCH * _CHW - _EPT
_NPT = 10240
_RPT = _NPT // _NT
_SUBR = 128
_NSUB = _RPT // _SUBR
_ROWS = 1000


def _mm_body(x_ref, w_ref, th_ref, o_ref, *, F, two_d_in):
    if two_d_in:
        x = jnp.concatenate([x_ref[0], x_ref[1]], axis=1)
    else:
        x = x_ref[...]
    wk = th_ref[0, 0, 0] * w_ref[:, 0:_HID]
    for h in range(1, _H):
        wk = wk + th_ref[0, 0, h] * w_ref[:, h * _HID:(h + 1) * _HID]
    y = jnp.dot(x, wk * (1.0 / _H), preferred_element_type=jnp.float32)
    o_ref[0, 0] = y[:, :_HALF]
    o_ref[0, 1] = y[:, _HALF:]


def _matmul_theta(h_in, W, theta):
    two_d = h_in.ndim == 3
    F = W.shape[0]
    if two_d:
        x_spec = pl.BlockSpec((2, _ROWS, _HALF), lambda k, i: (0, i, 0))
    else:
        x_spec = pl.BlockSpec((_ROWS, F), lambda k, i: (i, 0))
    return pl.pallas_call(
        functools.partial(_mm_body, F=F, two_d_in=two_d),
        grid=(_K, _N // _ROWS),
        in_specs=[
            x_spec,
            pl.BlockSpec((F, _H * _HID), lambda k, i: (0, 0)),
            pl.BlockSpec((1, 1, _H), lambda k, i: (k, 0, 0),
                         memory_space=pltpu.SMEM),
        ],
        out_specs=pl.BlockSpec((1, 2, _ROWS, _HALF), lambda k, i: (k, 0, i, 0)),
        out_shape=jax.ShapeDtypeStruct((_K, 2, _NPT, _HALF), jnp.float32),
    )(h_in, W, theta.reshape(_K, 1, _H))


def _head_body(h_ref, w1_ref, b1_ref, w2_ref, b2_ref, o_ref):
    g0 = jnp.sum(h_ref[0], axis=0, keepdims=True)
    g1 = jnp.sum(h_ref[1], axis=0, keepdims=True)
    g = jnp.concatenate([g0, g1], axis=1) * (1.0 / _N)
    g = jnp.maximum(jnp.dot(g, w1_ref[...], preferred_element_type=jnp.float32)
                    + b1_ref[...], 0.0)
    logits = jnp.dot(g, w2_ref[...], preferred_element_type=jnp.float32) + b2_ref[...]
    m = jnp.max(logits, axis=1, keepdims=True)
    s = logits - m
    o_ref[...] = s - jnp.log(jnp.sum(jnp.exp(s), axis=1, keepdims=True))


def _head(h2, lin1_W, lin1_b, lin2_W, lin2_b):
    C = lin2_W.shape[1]
    return pl.pallas_call(
        _head_body,
        out_shape=jax.ShapeDtypeStruct((1, C), jnp.float32),
    )(h2, lin1_W, lin1_b.reshape(1, -1), lin2_W, lin2_b.reshape(1, -1))


def _rsqrt_pos(d):
    i = lax.bitcast_convert_type(d, jnp.int32)
    i = jnp.int32(0x5F3759DF) - lax.shift_right_arithmetic(i, 1)
    y = lax.bitcast_convert_type(i, jnp.float32)
    for _ in range(4):
        y = y * (1.5 - 0.5 * d * y * y)
    return jnp.where(d > 0.5, y, 0.0)


def _sc_layer_body(first_layer, *refs):
    if first_layer:
        (y_hbm, ep_hbm, bias_hbm,
         h_out, dinv_out,
         CT, ACC,
         sbuf, gbuf, gbuf2, gbuf3, accb, ctb, ysub, dinvt, b0t, b1t, bb,
         ysem, ssem, gsem) = refs
    else:
        (y_hbm, ep_hbm, bias_hbm, dinv_hbm,
         h_out,
         CT, ACC,
         sbuf, gbuf, gbuf2, gbuf3, accb, ctb, ysub, dinvt, b0t, b1t, bb,
         ysem, ssem, gsem) = refs

    c = lax.axis_index("c")
    wid = lax.axis_index("s")
    row0 = wid * _RPT

    def vloop(n, body):
        lax.fori_loop(0, n, lambda i, _: (body(i), 0)[1], 0)

    def rh(i):
        return lax.div(i, jnp.int32(2)), pl.ds(16 * lax.rem(i, jnp.int32(2)), 16)

    def zero_accb(i):
        r, h16 = rh(i)
        accb[r, h16] = jnp.zeros((16,), jnp.float32)

    pltpu.sync_copy(bias_hbm.at[c], bb)
    vloop(2 * _SUBR, zero_accb)
    for s in range(_NSUB):
        pltpu.sync_copy(accb, ACC.at[pl.ds(row0 + s * _SUBR, _SUBR), :])
    if not first_layer:
        pltpu.sync_copy(dinv_hbm.at[pl.ds(row0, _RPT), :], dinvt)
    if first_layer:
        def fill_ones(i):
            gbuf[lax.div(i, jnp.int32(2)), pl.ds(16 * lax.rem(i, jnp.int32(2)), 16)] = (
                jnp.ones((16,), jnp.float32))
        vloop(2 * _CHW, fill_ones)
    plsc.subcore_barrier()

    def wait_scatter():
        pltpu.make_async_copy(gbuf, ACC.at[sbuf.at[0, 0]], ssem).wait()

    if first_layer:
        def deg_super(g, _):
            pltpu.sync_copy(ep_hbm.at[wid, pl.ds(g * _SCH, _SCH)], sbuf)
            for jj in range(_SCH):
                pltpu.async_copy(gbuf, ACC.at[sbuf.at[jj, 0]], ssem, add=True)
            for jj in range(_SCH):
                wait_scatter()
            return 0
        lax.fori_loop(0, _NSCH, deg_super, 0)
        plsc.subcore_barrier()

        def dinv_sub(s, _):
            r0 = row0 + s * _SUBR
            pltpu.sync_copy(ACC.at[pl.ds(r0, _SUBR), :], accb)

            def dinv_row(i):
                r, h16 = rh(i)
                dinvt[s * _SUBR + r, h16] = _rsqrt_pos(accb[r, h16])
                accb[r, h16] = jnp.zeros((16,), jnp.float32)
            vloop(2 * _SUBR, dinv_row)
            pltpu.sync_copy(accb, ACC.at[pl.ds(r0, _SUBR), :])
            return 0
        lax.fori_loop(0, _NSUB, dinv_sub, 0)

        @pl.when(c == 0)
        def _():
            pltpu.sync_copy(dinvt, dinv_out.at[pl.ds(row0, _RPT), :])

    def init_sub(s, _):
        r0 = row0 + s * _SUBR
        pltpu.sync_copy(y_hbm.at[_K - 1, c, pl.ds(r0, _SUBR), :], ysub)

        def init_row(i):
            r, h16 = rh(i)
            yv = ysub[r, h16]
            fl = pl.ds((s * _SUBR + r) * _HALF + 16 * lax.rem(i, jnp.int32(2)), 16)
            b1t[fl] = yv
            b0t[fl] = jnp.zeros((16,), jnp.float32)
            ctb[r, h16] = dinvt[s * _SUBR + r, h16] * yv
        vloop(2 * _SUBR, init_row)
        pltpu.sync_copy(ctb, CT.at[pl.ds(r0, _SUBR), :])
        return 0
    lax.fori_loop(0, _NSUB, init_sub, 0)
    plsc.subcore_barrier()

    def wait_gather(b):
        pltpu.make_async_copy(CT.at[sbuf.at[0, 1]], b, gsem).wait()

    def step(k, bprev, final):
        bufs = (gbuf, gbuf2, gbuf3)

        def super_chunk(g, _):
            pltpu.sync_copy(ep_hbm.at[wid, pl.ds(g * _SCH, _SCH)], sbuf)
            for jj in range(2):
                pltpu.async_copy(CT.at[sbuf.at[jj, 1]], bufs[jj], gsem)
            for jj in range(_SCH):
                b = bufs[jj % 3]
                wait_gather(b)
                pltpu.async_copy(b, ACC.at[sbuf.at[jj, 0]], ssem, add=True)
                if jj + 2 < _SCH:
                    if jj >= 1:
                        wait_scatter()
                    pltpu.async_copy(CT.at[sbuf.at[jj + 2, 1]],
                                     bufs[(jj + 2) % 3], gsem)
            for _ in range(3):
                wait_scatter()
            return 0
        lax.fori_loop(0, _NSCH, super_chunk, 0)
        plsc.subcore_barrier()

        def ew_sub(s, _):
            r0 = row0 + s * _SUBR
            pltpu.sync_copy(y_hbm.at[k, c, pl.ds(r0, _SUBR), :], ysub)
            pltpu.sync_copy(ACC.at[pl.ds(r0, _SUBR), :], accb)

            def ew_row(i):
                r, h16 = rh(i)
                fl = pl.ds((s * _SUBR + r) * _HALF + 16 * lax.rem(i, jnp.int32(2)), 16)
                a = accb[r, h16]
                d = dinvt[s * _SUBR + r, h16]
                yv = ysub[r, h16]
                if final:
                    out = yv - d * a - bprev[fl] + bb[pl.ds(16 * lax.rem(i, jnp.int32(2)), 16)]
                    ctb[r, h16] = jnp.maximum(out, 0.0)
                else:
                    bnew = yv - 2.0 * (d * a) - bprev[fl]
                    bprev[fl] = bnew
                    ctb[r, h16] = d * bnew
                    accb[r, h16] = jnp.zeros((16,), jnp.float32)
            vloop(2 * _SUBR, ew_row)

            if final:
                pltpu.sync_copy(ctb, h_out.at[c, pl.ds(r0, _SUBR), :])
            else:
                pltpu.sync_copy(ctb, CT.at[pl.ds(r0, _SUBR), :])
                pltpu.sync_copy(accb, ACC.at[pl.ds(r0, _SUBR), :])
            return 0
        lax.fori_loop(0, _NSUB, ew_sub, 0)
        if not final:
            plsc.subcore_barrier()

    def double_step(t, _):
        step(jnp.int32(12) - 2 * t, b0t, False)
        step(jnp.int32(11) - 2 * t, b1t, False)
        return 0
    lax.fori_loop(0, 6, double_step, 0)

    step(jnp.int32(0), b0t, True)

    @pl.when(wid == _NT - 1)
    def _():
        vloop(2 * _SUBR, zero_accb)
        for o in range(0, _NPT - _N - _SUBR + 1, _SUBR):
            pltpu.sync_copy(accb, h_out.at[c, pl.ds(_N + o, _SUBR), :])
        rem = (_NPT - _N) % _SUBR
        if rem:
            pltpu.sync_copy(accb.at[pl.ds(0, rem), :],
                            h_out.at[c, pl.ds(_NPT - rem, rem), :])


def _sc_layer(first_layer):
    out_type = [jax.ShapeDtypeStruct((2, _NPT, _HALF), jnp.float32)]
    if first_layer:
        out_type.append(jax.ShapeDtypeStruct((_NPT, _HALF), jnp.float32))
    return functools.partial(
        pl.kernel,
        functools.partial(_sc_layer_body, first_layer),
        out_type=tuple(out_type),
        mesh=plsc.VectorSubcoreMesh(core_axis_name="c", subcore_axis_name="s"),
        compiler_params=pltpu.CompilerParams(use_tc_tiling_on_sc=False),
        scratch_types=(
            pltpu.VMEM_SHARED((_NPT, _HALF), jnp.float32),
            pltpu.VMEM_SHARED((_NPT, _HALF), jnp.float32),
            pltpu.VMEM((_SCH, 2, _CHW), jnp.int32),
            pltpu.VMEM((_CHW, _HALF), jnp.float32),
            pltpu.VMEM((_CHW, _HALF), jnp.float32),
            pltpu.VMEM((_CHW, _HALF), jnp.float32),
            pltpu.VMEM((_SUBR, _HALF), jnp.float32),
            pltpu.VMEM((_SUBR, _HALF), jnp.float32),
            pltpu.VMEM((_SUBR, _HALF), jnp.float32),
            pltpu.VMEM((_RPT, _HALF), jnp.float32),
            pltpu.VMEM((_RPT * _HALF,), jnp.float32),
            pltpu.VMEM((_RPT * _HALF,), jnp.float32),
            pltpu.VMEM((_HALF,), jnp.float32),
            pltpu.SemaphoreType.DMA,
            pltpu.SemaphoreType.DMA,
            pltpu.SemaphoreType.DMA,
        ),
    )()


def kernel(x, edge_index, batch, W1, theta1, b1, W2, theta2, b2,
           lin1_W, lin1_b, lin2_W, lin2_b):
    ei = edge_index.reshape(2, _NT, _EPT)
    padv = jnp.full((2, _NT, _EPAD), _N, dtype=jnp.int32)
    ep = jnp.concatenate([ei, padv], axis=2).reshape(2, _NT, _NCH, _CHW)
    ep = jnp.transpose(ep, (1, 2, 0, 3))

    y1 = _matmul_theta(x, W1, theta1)
    h1, dinv = _sc_layer(True)(y1, ep, b1.reshape(2, _HALF))
    y2 = _matmul_theta(h1, W2, theta2)
    (h2,) = _sc_layer(False)(y2, ep, b2.reshape(2, _HALF), dinv)
    return _head(h2, lin1_W, lin1_b, lin2_W, lin2_b)


# speedup vs baseline: 326.7570x; 1.0637x over previous
import functools
import jax
import jax.numpy as jnp
from jax import lax
from jax.experimental import pallas as pl
from jax.experimental.pallas import tpu as pltpu
from jax.experimental.pallas import tpu_sc as plsc

_N = 10000
_E = 320000
_K = 14
_H = 8
_HID = 64
_HALF = 32
_NT = 16
_EPT = _E // _NT
_CHW = 128
_SCH = 8
_NCH = 160
_NSCH = _NCH // _SCH
_EPAD = _NCH * _CHW - _EPT
_NPT = 10240
_RPT = _NPT // _NT
_SUBR = 128
_NSUB = _RPT // _SUBR
_ROWS = 1000


def _mm_body(x_ref, w_ref, th_ref, o_ref, *, F, two_d_in):
    if two_d_in:
        x = jnp.concatenate([x_ref[0], x_ref[1]], axis=1)
    else:
        x = x_ref[...]
    wk = th_ref[0, 0, 0] * w_ref[:, 0:_HID]
    for h in range(1, _H):
        wk = wk + th_ref[0, 0, h] * w_ref[:, h * _HID:(h + 1) * _HID]
    y = jnp.dot(x, wk * (1.0 / _H), preferred_element_type=jnp.float32)
    o_ref[0, 0] = y[:, :_HALF]
    o_ref[0, 1] = y[:, _HALF:]


def _matmul_theta(h_in, W, theta):
    two_d = h_in.ndim == 3
    F = W.shape[0]
    if two_d:
        x_spec = pl.BlockSpec((2, _ROWS, _HALF), lambda k, i: (0, i, 0))
    else:
        x_spec = pl.BlockSpec((_ROWS, F), lambda k, i: (i, 0))
    return pl.pallas_call(
        functools.partial(_mm_body, F=F, two_d_in=two_d),
        grid=(_K, _N // _ROWS),
        in_specs=[
            x_spec,
            pl.BlockSpec((F, _H * _HID), lambda k, i: (0, 0)),
            pl.BlockSpec((1, 1, _H), lambda k, i: (k, 0, 0),
                         memory_space=pltpu.SMEM),
        ],
        out_specs=pl.BlockSpec((1, 2, _ROWS, _HALF), lambda k, i: (k, 0, i, 0)),
        out_shape=jax.ShapeDtypeStruct((_K, 2, _NPT, _HALF), jnp.float32),
    )(h_in, W, theta.reshape(_K, 1, _H))


def _head_body(h_ref, w1_ref, b1_ref, w2_ref, b2_ref, o_ref):
    g0 = jnp.sum(h_ref[0], axis=0, keepdims=True)
    g1 = jnp.sum(h_ref[1], axis=0, keepdims=True)
    g = jnp.concatenate([g0, g1], axis=1) * (1.0 / _N)
    g = jnp.maximum(jnp.dot(g, w1_ref[...], preferred_element_type=jnp.float32)
                    + b1_ref[...], 0.0)
    logits = jnp.dot(g, w2_ref[...], preferred_element_type=jnp.float32) + b2_ref[...]
    m = jnp.max(logits, axis=1, keepdims=True)
    s = logits - m
    o_ref[...] = s - jnp.log(jnp.sum(jnp.exp(s), axis=1, keepdims=True))


def _head(h2, lin1_W, lin1_b, lin2_W, lin2_b):
    C = lin2_W.shape[1]
    return pl.pallas_call(
        _head_body,
        out_shape=jax.ShapeDtypeStruct((1, C), jnp.float32),
    )(h2, lin1_W, lin1_b.reshape(1, -1), lin2_W, lin2_b.reshape(1, -1))


def _rsqrt_pos(d):
    i = lax.bitcast_convert_type(d, jnp.int32)
    i = jnp.int32(0x5F3759DF) - lax.shift_right_arithmetic(i, 1)
    y = lax.bitcast_convert_type(i, jnp.float32)
    for _ in range(4):
        y = y * (1.5 - 0.5 * d * y * y)
    return jnp.where(d > 0.5, y, 0.0)


def _sc_layer_body(first_layer, *refs):
    if first_layer:
        (y_hbm, ep_hbm, bias_hbm,
         h_out, dinv_out,
         CT, ACC,
         sbuf, sbuf2, gbuf, gbuf2, gbuf3, accb, ctb, ysub, dinvt, b0t, b1t, bb,
         ysem, ssem, gsem, isem) = refs
    else:
        (y_hbm, ep_hbm, bias_hbm, dinv_hbm,
         h_out,
         CT, ACC,
         sbuf, sbuf2, gbuf, gbuf2, gbuf3, accb, ctb, ysub, dinvt, b0t, b1t, bb,
         ysem, ssem, gsem, isem) = refs

    c = lax.axis_index("c")
    wid = lax.axis_index("s")
    row0 = wid * _RPT

    def vloop(n, body):
        lax.fori_loop(0, n, lambda i, _: (body(i), 0)[1], 0)

    def rh(i):
        return lax.div(i, jnp.int32(2)), pl.ds(16 * lax.rem(i, jnp.int32(2)), 16)

    def d16(r):
        return dinvt[r, pl.ds(0, 16)]

    def zero_accb(i):
        r, h16 = rh(i)
        accb[r, h16] = jnp.zeros((16,), jnp.float32)

    pltpu.sync_copy(bias_hbm.at[c], bb)
    vloop(2 * _SUBR, zero_accb)
    for s in range(_NSUB):
        pltpu.sync_copy(accb, ACC.at[pl.ds(row0 + s * _SUBR, _SUBR), :])
    if not first_layer:
        pltpu.sync_copy(dinv_hbm.at[pl.ds(row0, _RPT), :], dinvt)
    if first_layer:
        def fill_ones(i):
            gbuf[lax.div(i, jnp.int32(2)), pl.ds(16 * lax.rem(i, jnp.int32(2)), 16)] = (
                jnp.ones((16,), jnp.float32))
        vloop(2 * _CHW, fill_ones)
    plsc.subcore_barrier()

    def wait_scatter():
        pltpu.make_async_copy(gbuf, ACC.at[sbuf.at[0, 0]], ssem).wait()

    if first_layer:
        def deg_super(g, _):
            pltpu.sync_copy(ep_hbm.at[wid, pl.ds(g * _SCH, _SCH)], sbuf)
            for jj in range(_SCH):
                pltpu.async_copy(gbuf, ACC.at[sbuf.at[jj, 0]], ssem, add=True)
            for jj in range(_SCH):
                wait_scatter()
            return 0
        lax.fori_loop(0, _NSCH, deg_super, 0)
        plsc.subcore_barrier()

        def dinv_sub(s, _):
            r0 = row0 + s * _SUBR
            pltpu.sync_copy(ACC.at[pl.ds(r0, _SUBR), :], accb)

            def dinv_row(i):
                z = jnp.zeros((16,), jnp.float32)
                dinvt[s * _SUBR + i, pl.ds(0, 16)] = _rsqrt_pos(accb[i, pl.ds(0, 16)])
                accb[i, pl.ds(0, 16)] = z
                accb[i, pl.ds(16, 16)] = z
            vloop(_SUBR, dinv_row)
            pltpu.sync_copy(accb, ACC.at[pl.ds(r0, _SUBR), :])
            return 0
        lax.fori_loop(0, _NSUB, dinv_sub, 0)

        @pl.when(c == 0)
        def _():
            pltpu.sync_copy(dinvt, dinv_out.at[pl.ds(row0, _RPT), :])

    def init_sub(s, _):
        r0 = row0 + s * _SUBR
        pltpu.sync_copy(y_hbm.at[_K - 1, c, pl.ds(r0, _SUBR), :], ysub)

        def init_row(i):
            r, h16 = rh(i)
            yv = ysub[r, h16]
            fl = pl.ds((s * _SUBR + r) * _HALF + 16 * lax.rem(i, jnp.int32(2)), 16)
            b1t[fl] = yv
            b0t[fl] = jnp.zeros((16,), jnp.float32)
            ctb[r, h16] = d16(s * _SUBR + r) * yv
        vloop(2 * _SUBR, init_row)
        pltpu.sync_copy(ctb, CT.at[pl.ds(r0, _SUBR), :])
        return 0
    lax.fori_loop(0, _NSUB, init_sub, 0)
    plsc.subcore_barrier()

    def wait_gather(b):
        pltpu.make_async_copy(CT.at[sbuf.at[0, 1]], b, gsem).wait()

    def step(k, bprev, final):
        bufs = (gbuf, gbuf2, gbuf3)

        def proc_super(sb):
            for jj in range(2):
                pltpu.async_copy(CT.at[sb.at[jj, 1]], bufs[jj], gsem)
            for jj in range(_SCH):
                b = bufs[jj % 3]
                wait_gather(b)
                pltpu.async_copy(b, ACC.at[sb.at[jj, 0]], ssem, add=True)
                if jj + 2 < _SCH:
                    if jj >= 1:
                        wait_scatter()
                    pltpu.async_copy(CT.at[sb.at[jj + 2, 1]],
                                     bufs[(jj + 2) % 3], gsem)
            for _ in range(3):
                wait_scatter()

        def wait_idx():
            pltpu.make_async_copy(ep_hbm.at[wid, pl.ds(0, _SCH)], sbuf, isem).wait()

        pltpu.sync_copy(ep_hbm.at[wid, pl.ds(0, _SCH)], sbuf)

        def dbl_super(g2, _):
            pltpu.async_copy(ep_hbm.at[wid, pl.ds((2 * g2 + 1) * _SCH, _SCH)],
                             sbuf2, isem)
            proc_super(sbuf)
            wait_idx()

            @pl.when(g2 + 1 < _NSCH // 2)
            def _():
                pltpu.async_copy(ep_hbm.at[wid, pl.ds((2 * g2 + 2) * _SCH, _SCH)],
                                 sbuf, isem)
            proc_super(sbuf2)

            @pl.when(g2 + 1 < _NSCH // 2)
            def _():
                wait_idx()
            return 0
        lax.fori_loop(0, _NSCH // 2, dbl_super, 0)
        plsc.subcore_barrier()

        def ew_sub(s, _):
            r0 = row0 + s * _SUBR
            pltpu.sync_copy(y_hbm.at[k, c, pl.ds(r0, _SUBR), :], ysub)
            pltpu.sync_copy(ACC.at[pl.ds(r0, _SUBR), :], accb)

            def ew_row(i):
                r, h16 = rh(i)
                fl = pl.ds((s * _SUBR + r) * _HALF + 16 * lax.rem(i, jnp.int32(2)), 16)
                a = accb[r, h16]
                d = d16(s * _SUBR + r)
                yv = ysub[r, h16]
                if final:
                    out = yv - d * a - bprev[fl] + bb[pl.ds(16 * lax.rem(i, jnp.int32(2)), 16)]
                    ctb[r, h16] = jnp.maximum(out, 0.0)
                else:
                    bnew = yv - 2.0 * (d * a) - bprev[fl]
                    bprev[fl] = bnew
                    ctb[r, h16] = d * bnew
                    accb[r, h16] = jnp.zeros((16,), jnp.float32)
            vloop(2 * _SUBR, ew_row)

            if final:
                pltpu.sync_copy(ctb, h_out.at[c, pl.ds(r0, _SUBR), :])
            else:
                pltpu.sync_copy(ctb, CT.at[pl.ds(r0, _SUBR), :])
                pltpu.sync_copy(accb, ACC.at[pl.ds(r0, _SUBR), :])
            return 0
        lax.fori_loop(0, _NSUB, ew_sub, 0)
        if not final:
            plsc.subcore_barrier()

    def double_step(t, _):
        step(jnp.int32(12) - 2 * t, b0t, False)
        step(jnp.int32(11) - 2 * t, b1t, False)
        return 0
    lax.fori_loop(0, 6, double_step, 0)

    step(jnp.int32(0), b0t, True)

    @pl.when(wid == _NT - 1)
    def _():
        vloop(2 * _SUBR, zero_accb)
        for o in range(0, _NPT - _N - _SUBR + 1, _SUBR):
            pltpu.sync_copy(accb, h_out.at[c, pl.ds(_N + o, _SUBR), :])
        rem = (_NPT - _N) % _SUBR
        if rem:
            pltpu.sync_copy(accb.at[pl.ds(0, rem), :],
                            h_out.at[c, pl.ds(_NPT - rem, rem), :])


def _sc_layer(first_layer):
    out_type = [jax.ShapeDtypeStruct((2, _NPT, _HALF), jnp.float32)]
    if first_layer:
        out_type.append(jax.ShapeDtypeStruct((_NPT, 16), jnp.float32))
    return functools.partial(
        pl.kernel,
        functools.partial(_sc_layer_body, first_layer),
        out_type=tuple(out_type),
        mesh=plsc.VectorSubcoreMesh(core_axis_name="c", subcore_axis_name="s"),
        compiler_params=pltpu.CompilerParams(use_tc_tiling_on_sc=False),
        scratch_types=(
            pltpu.VMEM_SHARED((_NPT, _HALF), jnp.float32),
            pltpu.VMEM_SHARED((_NPT, _HALF), jnp.float32),
            pltpu.VMEM((_SCH, 2, _CHW), jnp.int32),
            pltpu.VMEM((_SCH, 2, _CHW), jnp.int32),
            pltpu.VMEM((_CHW, _HALF), jnp.float32),
            pltpu.VMEM((_CHW, _HALF), jnp.float32),
            pltpu.VMEM((_CHW, _HALF), jnp.float32),
            pltpu.VMEM((_SUBR, _HALF), jnp.float32),
            pltpu.VMEM((_SUBR, _HALF), jnp.float32),
            pltpu.VMEM((_SUBR, _HALF), jnp.float32),
            pltpu.VMEM((_RPT, 16), jnp.float32),
            pltpu.VMEM((_RPT * _HALF,), jnp.float32),
            pltpu.VMEM((_RPT * _HALF,), jnp.float32),
            pltpu.VMEM((_HALF,), jnp.float32),
            pltpu.SemaphoreType.DMA,
            pltpu.SemaphoreType.DMA,
            pltpu.SemaphoreType.DMA,
            pltpu.SemaphoreType.DMA,
        ),
    )()


def kernel(x, edge_index, batch, W1, theta1, b1, W2, theta2, b2,
           lin1_W, lin1_b, lin2_W, lin2_b):
    ei = edge_index.reshape(2, _NT, _EPT)
    padv = jnp.full((2, _NT, _EPAD), _N, dtype=jnp.int32)
    ep = jnp.concatenate([ei, padv], axis=2).reshape(2, _NT, _NCH, _CHW)
    ep = jnp.transpose(ep, (1, 2, 0, 3))

    y1 = _matmul_theta(x, W1, theta1)
    h1, dinv = _sc_layer(True)(y1, ep, b1.reshape(2, _HALF))
    y2 = _matmul_theta(h1, W2, theta2)
    (h2,) = _sc_layer(False)(y2, ep, b2.reshape(2, _HALF), dinv)
    return _head(h2, lin1_W, lin1_b, lin2_W, lin2_b)
